# baseline (speedup 1.0000x reference)
import jax
import jax.numpy as jnp
from jax.experimental import pallas as pl
from jax.experimental.pallas import tpu as pltpu


def _compiler_params(sem):
    return pltpu.CompilerParams(
        dimension_semantics=sem,
        vmem_limit_bytes=64 * 1024 * 1024,
    )


def _proj_kernel(x_ref, wa_ref, ws_ref, b_ref, p_ref, s_ref):
    xb = x_ref[...].astype(jnp.bfloat16)
    p_ref[...] = jnp.dot(xb, wa_ref[...],
                         preferred_element_type=jnp.float32).astype(jnp.bfloat16)
    s_ref[...] = jnp.dot(xb, ws_ref[...],
                         preferred_element_type=jnp.float32) + b_ref[...]


def _conv1_kernel(a_ref, p_ref, pself_ref, s_ref, dinv_ref,
                  w2a_ref, w2s_ref, b2_ref, q_ref, t_ref):
    agg = (jnp.dot(a_ref[...], p_ref[...], preferred_element_type=jnp.float32)
           + pself_ref[...].astype(jnp.float32))
    h1 = jnp.maximum(s_ref[...] + agg * dinv_ref[...], 0.0)
    h1b = h1.astype(jnp.bfloat16)
    q_ref[...] = jnp.dot(h1b, w2a_ref[...],
                         preferred_element_type=jnp.float32).astype(jnp.bfloat16)
    t_ref[...] = jnp.dot(h1b, w2s_ref[...],
                         preferred_element_type=jnp.float32) + b2_ref[...]


def _conv2_mlp_kernel(a_ref, q_ref, qself_ref, t_ref, dinv_ref,
                      wl1_ref, bl1_ref, wl2_ref, bl2_ref, wl3_ref, bl3_ref,
                      o_ref):
    agg = (jnp.dot(a_ref[...], q_ref[...], preferred_element_type=jnp.float32)
           + qself_ref[...].astype(jnp.float32))
    h2 = jnp.maximum(t_ref[...] + agg * dinv_ref[...], 0.0)
    s = jnp.dot(h2.astype(jnp.bfloat16), wl1_ref[...],
                preferred_element_type=jnp.float32) + bl1_ref[...]
    s = jnp.maximum(s, 0.0)
    s = jnp.dot(s.astype(jnp.bfloat16), wl2_ref[...],
                preferred_element_type=jnp.float32) + bl2_ref[...]
    s = jnp.maximum(s, 0.0)
    o_ref[...] = jnp.dot(s.astype(jnp.bfloat16), wl3_ref[...],
                         preferred_element_type=jnp.float32) + bl3_ref[...]


def _proj(x, wa1, ws1, b1, *, tm):
    n, f0 = x.shape
    f1 = wa1.shape[1]
    grid = (n // tm,)
    return pl.pallas_call(
        _proj_kernel,
        out_shape=(jax.ShapeDtypeStruct((n, f1), jnp.bfloat16),
                   jax.ShapeDtypeStruct((n, f1), jnp.float32)),
        grid=grid,
        in_specs=[
            pl.BlockSpec((tm, f0), lambda i: (i, 0)),
            pl.BlockSpec((f0, f1), lambda i: (0, 0)),
            pl.BlockSpec((f0, f1), lambda i: (0, 0)),
            pl.BlockSpec((1, f1), lambda i: (0, 0)),
        ],
        out_specs=(pl.BlockSpec((tm, f1), lambda i: (i, 0)),
                   pl.BlockSpec((tm, f1), lambda i: (i, 0))),
        compiler_params=_compiler_params(("parallel",)),
    )(x, wa1, ws1, b1)


def _conv1(a, p, s, dinv, w2a, w2s, b2, *, tm):
    n = a.shape[0]
    f1 = p.shape[1]
    f2 = w2a.shape[1]
    grid = (n // tm,)
    return pl.pallas_call(
        _conv1_kernel,
        out_shape=(jax.ShapeDtypeStruct((n, f2), jnp.bfloat16),
                   jax.ShapeDtypeStruct((n, f2), jnp.float32)),
        grid=grid,
        in_specs=[
            pl.BlockSpec((tm, n), lambda i: (i, 0)),
            pl.BlockSpec((n, f1), lambda i: (0, 0)),
            pl.BlockSpec((tm, f1), lambda i: (i, 0)),
            pl.BlockSpec((tm, f1), lambda i: (i, 0)),
            pl.BlockSpec((tm, 1), lambda i: (i, 0)),
            pl.BlockSpec((f1, f2), lambda i: (0, 0)),
            pl.BlockSpec((f1, f2), lambda i: (0, 0)),
            pl.BlockSpec((1, f2), lambda i: (0, 0)),
        ],
        out_specs=(pl.BlockSpec((tm, f2), lambda i: (i, 0)),
                   pl.BlockSpec((tm, f2), lambda i: (i, 0))),
        compiler_params=_compiler_params(("parallel",)),
    )(a, p, p, s, dinv, w2a, w2s, b2)


def _conv2_mlp(a, q, t, dinv, wl1, bl1, wl2, bl2, wl3, bl3, *, tm):
    n = a.shape[0]
    f2 = q.shape[1]
    l1 = wl1.shape[1]
    l2 = wl2.shape[1]
    l3 = wl3.shape[1]
    grid = (n // tm,)
    return pl.pallas_call(
        _conv2_mlp_kernel,
        out_shape=jax.ShapeDtypeStruct((n, l3), jnp.float32),
        grid=grid,
        in_specs=[
            pl.BlockSpec((tm, n), lambda i: (i, 0)),
            pl.BlockSpec((n, f2), lambda i: (0, 0)),
            pl.BlockSpec((tm, f2), lambda i: (i, 0)),
            pl.BlockSpec((tm, f2), lambda i: (i, 0)),
            pl.BlockSpec((tm, 1), lambda i: (i, 0)),
            pl.BlockSpec((f2, l1), lambda i: (0, 0)),
            pl.BlockSpec((1, l1), lambda i: (0, 0)),
            pl.BlockSpec((l1, l2), lambda i: (0, 0)),
            pl.BlockSpec((1, l2), lambda i: (0, 0)),
            pl.BlockSpec((l2, l3), lambda i: (0, 0)),
            pl.BlockSpec((1, l3), lambda i: (0, 0)),
        ],
        out_specs=pl.BlockSpec((tm, l3), lambda i: (i, 0)),
        compiler_params=_compiler_params(("parallel",)),
    )(a, q, q, t, dinv, wl1, bl1, wl2, bl2, wl3, bl3)


def kernel(x, edge_index, ws1, wa1, b1, ws2, wa2, b2,
           wl1, bl1, wl2, bl2, wl3, bl3):
    n = x.shape[0]
    tm = 1024

    src, dst = edge_index[0], edge_index[1]
    not_self = src != dst
    a = jnp.zeros((n, n), jnp.bfloat16).at[dst, src].add(
        not_self.astype(jnp.bfloat16))
    deg = jnp.zeros((n,), jnp.float32).at[dst].add(
        not_self.astype(jnp.float32)) + 1.0
    dinv = (1.0 / deg)[:, None]

    p, s = _proj(x, wa1, ws1, b1, tm=tm)
    q, t = _conv1(a, p, s, dinv, wa2, ws2, b2, tm=tm)
    out = _conv2_mlp(a, q, t, dinv, wl1, bl1, wl2, bl2, wl3, bl3, tm=tm)
    return out[:, 0]


# speedup vs baseline: 1.4143x; 1.4143x over previous
import jax
import jax.numpy as jnp
from jax.experimental import pallas as pl
from jax.experimental.pallas import tpu as pltpu


def _compiler_params(sem):
    return pltpu.CompilerParams(
        dimension_semantics=sem,
        vmem_limit_bytes=64 * 1024 * 1024,
    )


def _proj_kernel(x_ref, wa_ref, ws_ref, b_ref, p_ref, s_ref):
    xb = x_ref[...].astype(jnp.bfloat16)
    p_ref[...] = jnp.dot(xb, wa_ref[...],
                         preferred_element_type=jnp.float32).astype(jnp.bfloat16)
    s_ref[...] = jnp.dot(xb, ws_ref[...],
                         preferred_element_type=jnp.float32) + b_ref[...]


def _conv1_kernel(a_ref, p_ref, pself_ref, s_ref, dinv_ref,
                  w2a_ref, w2s_ref, b2_ref, q_ref, t_ref):
    agg = (jnp.dot(a_ref[...], p_ref[...], preferred_element_type=jnp.float32)
           + pself_ref[...].astype(jnp.float32))
    h1 = jnp.maximum(s_ref[...] + agg * dinv_ref[...], 0.0)
    h1b = h1.astype(jnp.bfloat16)
    q_ref[...] = jnp.dot(h1b, w2a_ref[...],
                         preferred_element_type=jnp.float32).astype(jnp.bfloat16)
    t_ref[...] = jnp.dot(h1b, w2s_ref[...],
                         preferred_element_type=jnp.float32) + b2_ref[...]


def _conv2_mlp_kernel(a_ref, q_ref, qself_ref, t_ref, dinv_ref,
                      wl1_ref, bl1_ref, wl2_ref, bl2_ref, wl3_ref, bl3_ref,
                      o_ref):
    agg = (jnp.dot(a_ref[...], q_ref[...], preferred_element_type=jnp.float32)
           + qself_ref[...].astype(jnp.float32))
    h2 = jnp.maximum(t_ref[...] + agg * dinv_ref[...], 0.0)
    s = jnp.dot(h2.astype(jnp.bfloat16), wl1_ref[...],
                preferred_element_type=jnp.float32) + bl1_ref[...]
    s = jnp.maximum(s, 0.0)
    s = jnp.dot(s.astype(jnp.bfloat16), wl2_ref[...],
                preferred_element_type=jnp.float32) + bl2_ref[...]
    s = jnp.maximum(s, 0.0)
    o_ref[...] = jnp.dot(s.astype(jnp.bfloat16), wl3_ref[...],
                         preferred_element_type=jnp.float32) + bl3_ref[...]


def _proj(x, wa1, ws1, b1, *, tm):
    n, f0 = x.shape
    f1 = wa1.shape[1]
    grid = (n // tm,)
    return pl.pallas_call(
        _proj_kernel,
        out_shape=(jax.ShapeDtypeStruct((n, f1), jnp.bfloat16),
                   jax.ShapeDtypeStruct((n, f1), jnp.float32)),
        grid=grid,
        in_specs=[
            pl.BlockSpec((tm, f0), lambda i: (i, 0)),
            pl.BlockSpec((f0, f1), lambda i: (0, 0)),
            pl.BlockSpec((f0, f1), lambda i: (0, 0)),
            pl.BlockSpec((1, f1), lambda i: (0, 0)),
        ],
        out_specs=(pl.BlockSpec((tm, f1), lambda i: (i, 0)),
                   pl.BlockSpec((tm, f1), lambda i: (i, 0))),
        compiler_params=_compiler_params(("parallel",)),
    )(x, wa1, ws1, b1)


def _conv1(a, p, s, dinv, w2a, w2s, b2, *, tm):
    n = a.shape[0]
    f1 = p.shape[1]
    f2 = w2a.shape[1]
    grid = (n // tm,)
    return pl.pallas_call(
        _conv1_kernel,
        out_shape=(jax.ShapeDtypeStruct((n, f2), jnp.bfloat16),
                   jax.ShapeDtypeStruct((n, f2), jnp.float32)),
        grid=grid,
        in_specs=[
            pl.BlockSpec((tm, n), lambda i: (i, 0)),
            pl.BlockSpec((n, f1), lambda i: (0, 0)),
            pl.BlockSpec((tm, f1), lambda i: (i, 0)),
            pl.BlockSpec((tm, f1), lambda i: (i, 0)),
            pl.BlockSpec((tm, 1), lambda i: (i, 0)),
            pl.BlockSpec((f1, f2), lambda i: (0, 0)),
            pl.BlockSpec((f1, f2), lambda i: (0, 0)),
            pl.BlockSpec((1, f2), lambda i: (0, 0)),
        ],
        out_specs=(pl.BlockSpec((tm, f2), lambda i: (i, 0)),
                   pl.BlockSpec((tm, f2), lambda i: (i, 0))),
        compiler_params=_compiler_params(("parallel",)),
    )(a, p, p, s, dinv, w2a, w2s, b2)


def _conv2_mlp(a, q, t, dinv, wl1, bl1, wl2, bl2, wl3, bl3, *, tm):
    n = a.shape[0]
    f2 = q.shape[1]
    l1 = wl1.shape[1]
    l2 = wl2.shape[1]
    l3 = wl3.shape[1]
    grid = (n // tm,)
    return pl.pallas_call(
        _conv2_mlp_kernel,
        out_shape=jax.ShapeDtypeStruct((n, l3), jnp.float32),
        grid=grid,
        in_specs=[
            pl.BlockSpec((tm, n), lambda i: (i, 0)),
            pl.BlockSpec((n, f2), lambda i: (0, 0)),
            pl.BlockSpec((tm, f2), lambda i: (i, 0)),
            pl.BlockSpec((tm, f2), lambda i: (i, 0)),
            pl.BlockSpec((tm, 1), lambda i: (i, 0)),
            pl.BlockSpec((f2, l1), lambda i: (0, 0)),
            pl.BlockSpec((1, l1), lambda i: (0, 0)),
            pl.BlockSpec((l1, l2), lambda i: (0, 0)),
            pl.BlockSpec((1, l2), lambda i: (0, 0)),
            pl.BlockSpec((l2, l3), lambda i: (0, 0)),
            pl.BlockSpec((1, l3), lambda i: (0, 0)),
        ],
        out_specs=pl.BlockSpec((tm, l3), lambda i: (i, 0)),
        compiler_params=_compiler_params(("parallel",)),
    )(a, q, q, t, dinv, wl1, bl1, wl2, bl2, wl3, bl3)


def kernel(x, edge_index, ws1, wa1, b1, ws2, wa2, b2,
           wl1, bl1, wl2, bl2, wl3, bl3):
    n = x.shape[0]
    tm = 1024

    src, dst = edge_index[0], edge_index[1]
    not_self = src != dst
    a = jnp.zeros((n, n), jnp.float32).at[dst, src].add(
        not_self.astype(jnp.float32)).astype(jnp.bfloat16)
    deg = jnp.zeros((n,), jnp.float32).at[dst].add(
        not_self.astype(jnp.float32)) + 1.0
    dinv = (1.0 / deg)[:, None]

    p, s = _proj(x, wa1, ws1, b1, tm=tm)
    q, t = _conv1(a, p, s, dinv, wa2, ws2, b2, tm=tm)
    out = _conv2_mlp(a, q, t, dinv, wl1, bl1, wl2, bl2, wl3, bl3, tm=tm)
    return out[:, 0]


# speedup vs baseline: 1.9484x; 1.3776x over previous
import jax
import jax.numpy as jnp
from jax.experimental import pallas as pl
from jax.experimental.pallas import tpu as pltpu

LANE = 128


def _compiler_params(sem):
    return pltpu.CompilerParams(
        dimension_semantics=sem,
        vmem_limit_bytes=64 * 1024 * 1024,
    )


def _proj_kernel(x_ref, wa_ref, ws_ref, b_ref, p_ref, s_ref):
    xb = x_ref[...].astype(jnp.bfloat16)
    p_ref[...] = jnp.dot(xb, wa_ref[...],
                         preferred_element_type=jnp.float32).astype(jnp.bfloat16)
    s_ref[...] = jnp.dot(xb, ws_ref[...],
                         preferred_element_type=jnp.float32) + b_ref[...]


def _agg_dot(a_ref, p3_ref):
    kk = a_ref.shape[0]
    lhs = jnp.concatenate(
        [a_ref[j].astype(jnp.bfloat16) for j in range(kk)], axis=1)
    rhs = jnp.concatenate([p3_ref[j] for j in range(kk)], axis=0)
    return jnp.dot(lhs, rhs, preferred_element_type=jnp.float32)


def _conv1_kernel(a_ref, p3_ref, pself_ref, s_ref, dinv_ref,
                  w2a_ref, w2s_ref, b2_ref, q_ref, t_ref, acc_ref):
    k = pl.program_id(1)

    @pl.when(k == 0)
    def _():
        acc_ref[...] = pself_ref[...].astype(jnp.float32)

    acc_ref[...] += _agg_dot(a_ref, p3_ref)

    @pl.when(k == pl.num_programs(1) - 1)
    def _():
        h1 = jnp.maximum(s_ref[...] + acc_ref[...] * dinv_ref[...], 0.0)
        h1b = h1.astype(jnp.bfloat16)
        q_ref[...] = jnp.dot(h1b, w2a_ref[...],
                             preferred_element_type=jnp.float32).astype(jnp.bfloat16)
        t_ref[...] = jnp.dot(h1b, w2s_ref[...],
                             preferred_element_type=jnp.float32) + b2_ref[...]


def _conv2_mlp_kernel(a_ref, q3_ref, qself_ref, t_ref, dinv_ref,
                      wl1_ref, bl1_ref, wl2_ref, bl2_ref, wl3_ref, bl3_ref,
                      o_ref, acc_ref):
    k = pl.program_id(1)

    @pl.when(k == 0)
    def _():
        acc_ref[...] = qself_ref[...].astype(jnp.float32)

    acc_ref[...] += _agg_dot(a_ref, q3_ref)

    @pl.when(k == pl.num_programs(1) - 1)
    def _():
        h2 = jnp.maximum(t_ref[...] + acc_ref[...] * dinv_ref[...], 0.0)
        s = jnp.dot(h2.astype(jnp.bfloat16), wl1_ref[...],
                    preferred_element_type=jnp.float32) + bl1_ref[...]
        s = jnp.maximum(s, 0.0)
        s = jnp.dot(s.astype(jnp.bfloat16), wl2_ref[...],
                    preferred_element_type=jnp.float32) + bl2_ref[...]
        s = jnp.maximum(s, 0.0)
        o_ref[...] = jnp.dot(s.astype(jnp.bfloat16), wl3_ref[...],
                             preferred_element_type=jnp.float32) + bl3_ref[...]


def _proj(x, wa1, ws1, b1, *, tm):
    n, f0 = x.shape
    f1 = wa1.shape[1]
    grid = (n // tm,)
    return pl.pallas_call(
        _proj_kernel,
        out_shape=(jax.ShapeDtypeStruct((n, f1), jnp.bfloat16),
                   jax.ShapeDtypeStruct((n, f1), jnp.float32)),
        grid=grid,
        in_specs=[
            pl.BlockSpec((tm, f0), lambda i: (i, 0)),
            pl.BlockSpec((f0, f1), lambda i: (0, 0)),
            pl.BlockSpec((f0, f1), lambda i: (0, 0)),
            pl.BlockSpec((1, f1), lambda i: (0, 0)),
        ],
        out_specs=(pl.BlockSpec((tm, f1), lambda i: (i, 0)),
                   pl.BlockSpec((tm, f1), lambda i: (i, 0))),
        compiler_params=_compiler_params(("parallel",)),
    )(x, wa1, ws1, b1)


def _conv1(a3, p, s, dinv, w2a, w2s, b2, *, tm, kk):
    ng, n, _ = a3.shape
    f1 = p.shape[1]
    f2 = w2a.shape[1]
    p3 = p.reshape(ng, LANE, f1)
    grid = (n // tm, ng // kk)
    return pl.pallas_call(
        _conv1_kernel,
        out_shape=(jax.ShapeDtypeStruct((n, f2), jnp.bfloat16),
                   jax.ShapeDtypeStruct((n, f2), jnp.float32)),
        grid=grid,
        in_specs=[
            pl.BlockSpec((kk, tm, LANE), lambda i, k: (k, i, 0)),
            pl.BlockSpec((kk, LANE, f1), lambda i, k: (k, 0, 0)),
            pl.BlockSpec((tm, f1), lambda i, k: (i, 0)),
            pl.BlockSpec((tm, f1), lambda i, k: (i, 0)),
            pl.BlockSpec((tm, 1), lambda i, k: (i, 0)),
            pl.BlockSpec((f1, f2), lambda i, k: (0, 0)),
            pl.BlockSpec((f1, f2), lambda i, k: (0, 0)),
            pl.BlockSpec((1, f2), lambda i, k: (0, 0)),
        ],
        out_specs=(pl.BlockSpec((tm, f2), lambda i, k: (i, 0)),
                   pl.BlockSpec((tm, f2), lambda i, k: (i, 0))),
        scratch_shapes=[pltpu.VMEM((tm, f2), jnp.float32)],
        compiler_params=_compiler_params(("parallel", "arbitrary")),
    )(a3, p3, p, s, dinv, w2a, w2s, b2)


def _conv2_mlp(a3, q, t, dinv, wl1, bl1, wl2, bl2, wl3, bl3, *, tm, kk):
    ng, n, _ = a3.shape
    f2 = q.shape[1]
    l1 = wl1.shape[1]
    l2 = wl2.shape[1]
    l3 = wl3.shape[1]
    q3 = q.reshape(ng, LANE, f2)
    grid = (n // tm, ng // kk)
    return pl.pallas_call(
        _conv2_mlp_kernel,
        out_shape=jax.ShapeDtypeStruct((n, l3), jnp.float32),
        grid=grid,
        in_specs=[
            pl.BlockSpec((kk, tm, LANE), lambda i, k: (k, i, 0)),
            pl.BlockSpec((kk, LANE, f2), lambda i, k: (k, 0, 0)),
            pl.BlockSpec((tm, f2), lambda i, k: (i, 0)),
            pl.BlockSpec((tm, f2), lambda i, k: (i, 0)),
            pl.BlockSpec((tm, 1), lambda i, k: (i, 0)),
            pl.BlockSpec((f2, l1), lambda i, k: (0, 0)),
            pl.BlockSpec((1, l1), lambda i, k: (0, 0)),
            pl.BlockSpec((l1, l2), lambda i, k: (0, 0)),
            pl.BlockSpec((1, l2), lambda i, k: (0, 0)),
            pl.BlockSpec((l2, l3), lambda i, k: (0, 0)),
            pl.BlockSpec((1, l3), lambda i, k: (0, 0)),
        ],
        out_specs=pl.BlockSpec((tm, l3), lambda i, k: (i, 0)),
        scratch_shapes=[pltpu.VMEM((tm, f2), jnp.float32)],
        compiler_params=_compiler_params(("parallel", "arbitrary")),
    )(a3, q3, q, t, dinv, wl1, bl1, wl2, bl2, wl3, bl3)


def kernel(x, edge_index, ws1, wa1, b1, ws2, wa2, b2,
           wl1, bl1, wl2, bl2, wl3, bl3):
    n = x.shape[0]
    ng = n // LANE
    tm, kk = 1024, 16

    src, dst = edge_index[0], edge_index[1]
    not_self = src != dst

    lin = ((src >> 7) * (n * LANE)
           + (dst >> 3) * 1024
           + (dst & 7) * LANE
           + (src & 127))
    a3 = jnp.zeros((ng * n * LANE,), jnp.float32).at[lin].add(
        not_self.astype(jnp.float32)).reshape(ng, n, LANE)

    deg = jnp.zeros((n,), jnp.float32).at[dst].add(
        not_self.astype(jnp.float32)) + 1.0
    dinv = (1.0 / deg)[:, None]

    p, s = _proj(x, wa1, ws1, b1, tm=tm)
    q, t = _conv1(a3, p, s, dinv, wa2, ws2, b2, tm=tm, kk=kk)
    out = _conv2_mlp(a3, q, t, dinv, wl1, bl1, wl2, bl2, wl3, bl3,
                     tm=tm, kk=kk)
    return out[:, 0]


# speedup vs baseline: 2.1710x; 1.1143x over previous
import jax
import jax.numpy as jnp
from jax.experimental import pallas as pl
from jax.experimental.pallas import tpu as pltpu

LANE = 128


def _compiler_params(sem):
    return pltpu.CompilerParams(
        dimension_semantics=sem,
        vmem_limit_bytes=64 * 1024 * 1024,
    )


def _proj_kernel(x_ref, wa_ref, ws_ref, b_ref, p_ref, s_ref):
    xb = x_ref[...].astype(jnp.bfloat16)
    pb = jnp.dot(xb, wa_ref[...],
                 preferred_element_type=jnp.float32).astype(jnp.bfloat16)
    tm = pb.shape[0]
    ones_col = (jax.lax.broadcasted_iota(jnp.int32, (tm, LANE), 1)
                == 0).astype(jnp.bfloat16)
    p_ref[...] = jnp.concatenate([pb, ones_col], axis=1)
    s_ref[...] = jnp.dot(xb, ws_ref[...],
                         preferred_element_type=jnp.float32) + b_ref[...]


def _agg_dot(a_ref, p3_ref):
    kk = a_ref.shape[0]
    lhs = jnp.concatenate(
        [a_ref[j].astype(jnp.bfloat16) for j in range(kk)], axis=1)
    rhs = jnp.concatenate([p3_ref[j] for j in range(kk)], axis=0)
    return jnp.dot(lhs, rhs, preferred_element_type=jnp.float32)


def _conv1_kernel(a_ref, p3_ref, pself_ref, s_ref,
                  w2a_ref, w2s_ref, b2_ref, q_ref, t_ref, dinv_ref, acc_ref):
    k = pl.program_id(1)

    @pl.when(k == 0)
    def _():
        acc_ref[...] = pself_ref[...].astype(jnp.float32)

    acc_ref[...] += _agg_dot(a_ref, p3_ref)

    @pl.when(k == pl.num_programs(1) - 1)
    def _():
        f1 = s_ref.shape[1]
        acc = acc_ref[...]
        dinv = 1.0 / acc[:, f1:f1 + 1]
        h1 = jnp.maximum(s_ref[...] + acc[:, :f1] * dinv, 0.0)
        h1b = h1.astype(jnp.bfloat16)
        q_ref[...] = jnp.dot(h1b, w2a_ref[...],
                             preferred_element_type=jnp.float32).astype(jnp.bfloat16)
        t_ref[...] = jnp.dot(h1b, w2s_ref[...],
                             preferred_element_type=jnp.float32) + b2_ref[...]
        dinv_ref[...] = dinv


def _conv2_mlp_kernel(a_ref, q3_ref, qself_ref, t_ref, dinv_ref,
                      wl1_ref, bl1_ref, wl2_ref, bl2_ref, wl3_ref, bl3_ref,
                      o_ref, acc_ref):
    k = pl.program_id(1)

    @pl.when(k == 0)
    def _():
        acc_ref[...] = qself_ref[...].astype(jnp.float32)

    acc_ref[...] += _agg_dot(a_ref, q3_ref)

    @pl.when(k == pl.num_programs(1) - 1)
    def _():
        h2 = jnp.maximum(t_ref[...] + acc_ref[...] * dinv_ref[...], 0.0)
        s = jnp.dot(h2.astype(jnp.bfloat16), wl1_ref[...],
                    preferred_element_type=jnp.float32) + bl1_ref[...]
        s = jnp.maximum(s, 0.0)
        s = jnp.dot(s.astype(jnp.bfloat16), wl2_ref[...],
                    preferred_element_type=jnp.float32) + bl2_ref[...]
        s = jnp.maximum(s, 0.0)
        o_ref[...] = jnp.dot(s.astype(jnp.bfloat16), wl3_ref[...],
                             preferred_element_type=jnp.float32) + bl3_ref[...]


def _proj(x, wa1, ws1, b1, *, tm):
    n, f0 = x.shape
    f1 = wa1.shape[1]
    grid = (n // tm,)
    return pl.pallas_call(
        _proj_kernel,
        out_shape=(jax.ShapeDtypeStruct((n, f1 + LANE), jnp.bfloat16),
                   jax.ShapeDtypeStruct((n, f1), jnp.float32)),
        grid=grid,
        in_specs=[
            pl.BlockSpec((tm, f0), lambda i: (i, 0)),
            pl.BlockSpec((f0, f1), lambda i: (0, 0)),
            pl.BlockSpec((f0, f1), lambda i: (0, 0)),
            pl.BlockSpec((1, f1), lambda i: (0, 0)),
        ],
        out_specs=(pl.BlockSpec((tm, f1 + LANE), lambda i: (i, 0)),
                   pl.BlockSpec((tm, f1), lambda i: (i, 0))),
        compiler_params=_compiler_params(("parallel",)),
    )(x, wa1, ws1, b1)


def _conv1(a3, p, s, w2a, w2s, b2, *, tm, kk):
    ng, n, _ = a3.shape
    f1a = p.shape[1]
    f1 = s.shape[1]
    f2 = w2a.shape[1]
    p3 = p.reshape(ng, LANE, f1a)
    grid = (n // tm, ng // kk)
    return pl.pallas_call(
        _conv1_kernel,
        out_shape=(jax.ShapeDtypeStruct((n, f2), jnp.bfloat16),
                   jax.ShapeDtypeStruct((n, f2), jnp.float32),
                   jax.ShapeDtypeStruct((n, 1), jnp.float32)),
        grid=grid,
        in_specs=[
            pl.BlockSpec((kk, tm, LANE), lambda i, k: (k, i, 0)),
            pl.BlockSpec((kk, LANE, f1a), lambda i, k: (k, 0, 0)),
            pl.BlockSpec((tm, f1a), lambda i, k: (i, 0)),
            pl.BlockSpec((tm, f1), lambda i, k: (i, 0)),
            pl.BlockSpec((f1, f2), lambda i, k: (0, 0)),
            pl.BlockSpec((f1, f2), lambda i, k: (0, 0)),
            pl.BlockSpec((1, f2), lambda i, k: (0, 0)),
        ],
        out_specs=(pl.BlockSpec((tm, f2), lambda i, k: (i, 0)),
                   pl.BlockSpec((tm, f2), lambda i, k: (i, 0)),
                   pl.BlockSpec((tm, 1), lambda i, k: (i, 0))),
        scratch_shapes=[pltpu.VMEM((tm, f1a), jnp.float32)],
        compiler_params=_compiler_params(("parallel", "arbitrary")),
    )(a3, p3, p, s, w2a, w2s, b2)


def _conv2_mlp(a3, q, t, dinv, wl1, bl1, wl2, bl2, wl3, bl3, *, tm, kk):
    ng, n, _ = a3.shape
    f2 = q.shape[1]
    l1 = wl1.shape[1]
    l2 = wl2.shape[1]
    l3 = wl3.shape[1]
    q3 = q.reshape(ng, LANE, f2)
    grid = (n // tm, ng // kk)
    return pl.pallas_call(
        _conv2_mlp_kernel,
        out_shape=jax.ShapeDtypeStruct((n, l3), jnp.float32),
        grid=grid,
        in_specs=[
            pl.BlockSpec((kk, tm, LANE), lambda i, k: (k, i, 0)),
            pl.BlockSpec((kk, LANE, f2), lambda i, k: (k, 0, 0)),
            pl.BlockSpec((tm, f2), lambda i, k: (i, 0)),
            pl.BlockSpec((tm, f2), lambda i, k: (i, 0)),
            pl.BlockSpec((tm, 1), lambda i, k: (i, 0)),
            pl.BlockSpec((f2, l1), lambda i, k: (0, 0)),
            pl.BlockSpec((1, l1), lambda i, k: (0, 0)),
            pl.BlockSpec((l1, l2), lambda i, k: (0, 0)),
            pl.BlockSpec((1, l2), lambda i, k: (0, 0)),
            pl.BlockSpec((l2, l3), lambda i, k: (0, 0)),
            pl.BlockSpec((1, l3), lambda i, k: (0, 0)),
        ],
        out_specs=pl.BlockSpec((tm, l3), lambda i, k: (i, 0)),
        scratch_shapes=[pltpu.VMEM((tm, f2), jnp.float32)],
        compiler_params=_compiler_params(("parallel", "arbitrary")),
    )(a3, q3, q, t, dinv, wl1, bl1, wl2, bl2, wl3, bl3)


def kernel(x, edge_index, ws1, wa1, b1, ws2, wa2, b2,
           wl1, bl1, wl2, bl2, wl3, bl3):
    n = x.shape[0]
    ng = n // LANE
    tm, kk = 1024, 8

    src, dst = edge_index[0], edge_index[1]
    not_self = src != dst

    lin = ((src >> 7) * (n * LANE)
           + (dst >> 3) * 1024
           + (dst & 7) * LANE
           + (src & 127))
    a3 = jnp.zeros((ng * n * LANE,), jnp.float32).at[lin].add(
        not_self.astype(jnp.float32)).reshape(ng, n, LANE)

    p, s = _proj(x, wa1, ws1, b1, tm=tm)
    q, t, dinv = _conv1(a3, p, s, wa2, ws2, b2, tm=tm, kk=kk)
    out = _conv2_mlp(a3, q, t, dinv, wl1, bl1, wl2, bl2, wl3, bl3,
                     tm=tm, kk=kk)
    return out[:, 0]


# speedup vs baseline: 2.4361x; 1.1221x over previous
import jax
import jax.numpy as jnp
from jax.experimental import pallas as pl
from jax.experimental.pallas import tpu as pltpu

LANE = 128


def _compiler_params(sem):
    return pltpu.CompilerParams(
        dimension_semantics=sem,
        vmem_limit_bytes=64 * 1024 * 1024,
    )


def _proj_kernel(x_ref, wa_ref, ws_ref, b_ref, p_ref, s_ref):
    xb = x_ref[...].astype(jnp.bfloat16)
    pb = jnp.dot(xb, wa_ref[...],
                 preferred_element_type=jnp.float32).astype(jnp.bfloat16)
    tm = pb.shape[0]
    ones_col = (jax.lax.broadcasted_iota(jnp.int32, (tm, LANE), 1)
                == 0).astype(jnp.bfloat16)
    p_ref[...] = jnp.concatenate([pb, ones_col], axis=1)
    s_ref[...] = jnp.dot(xb, ws_ref[...],
                         preferred_element_type=jnp.float32) + b_ref[...]


def _agg_dot(a_ref, p3_ref):
    kk = a_ref.shape[0]
    lhs = jnp.concatenate(
        [a_ref[j].astype(jnp.bfloat16) for j in range(kk)], axis=1)
    rhs = jnp.concatenate([p3_ref[j] for j in range(kk)], axis=0)
    return jnp.dot(lhs, rhs, preferred_element_type=jnp.float32)


def _conv1_kernel(a_ref, p3_ref, pself_ref, s_ref,
                  w2a_ref, w2s_ref, b2_ref, q_ref, t_ref, dinv_ref):
    f1 = s_ref.shape[1]
    acc = _agg_dot(a_ref, p3_ref) + pself_ref[...].astype(jnp.float32)
    dinv = 1.0 / acc[:, f1:f1 + 1]
    h1 = jnp.maximum(s_ref[...] + acc[:, :f1] * dinv, 0.0)
    h1b = h1.astype(jnp.bfloat16)
    q_ref[...] = jnp.dot(h1b, w2a_ref[...],
                         preferred_element_type=jnp.float32).astype(jnp.bfloat16)
    t_ref[...] = jnp.dot(h1b, w2s_ref[...],
                         preferred_element_type=jnp.float32) + b2_ref[...]
    dinv_ref[...] = dinv


def _conv2_mlp_kernel(a_ref, q3_ref, qself_ref, t_ref, dinv_ref,
                      wl1_ref, bl1_ref, wl2_ref, bl2_ref, wl3_ref, bl3_ref,
                      o_ref):
    acc = _agg_dot(a_ref, q3_ref) + qself_ref[...].astype(jnp.float32)
    h2 = jnp.maximum(t_ref[...] + acc * dinv_ref[...], 0.0)
    s = jnp.dot(h2.astype(jnp.bfloat16), wl1_ref[...],
                preferred_element_type=jnp.float32) + bl1_ref[...]
    s = jnp.maximum(s, 0.0)
    s = jnp.dot(s.astype(jnp.bfloat16), wl2_ref[...],
                preferred_element_type=jnp.float32) + bl2_ref[...]
    s = jnp.maximum(s, 0.0)
    o_ref[...] = jnp.dot(s.astype(jnp.bfloat16), wl3_ref[...],
                         preferred_element_type=jnp.float32) + bl3_ref[...]


def _proj(x, wa1, ws1, b1, *, tm):
    n, f0 = x.shape
    f1 = wa1.shape[1]
    grid = (n // tm,)
    return pl.pallas_call(
        _proj_kernel,
        out_shape=(jax.ShapeDtypeStruct((n, f1 + LANE), jnp.bfloat16),
                   jax.ShapeDtypeStruct((n, f1), jnp.float32)),
        grid=grid,
        in_specs=[
            pl.BlockSpec((tm, f0), lambda i: (i, 0)),
            pl.BlockSpec((f0, f1), lambda i: (0, 0)),
            pl.BlockSpec((f0, f1), lambda i: (0, 0)),
            pl.BlockSpec((1, f1), lambda i: (0, 0)),
        ],
        out_specs=(pl.BlockSpec((tm, f1 + LANE), lambda i: (i, 0)),
                   pl.BlockSpec((tm, f1), lambda i: (i, 0))),
        compiler_params=_compiler_params(("parallel",)),
    )(x, wa1, ws1, b1)


def _conv1(a3, p, s, w2a, w2s, b2, *, tm, kk):
    ng, n, _ = a3.shape
    f1a = p.shape[1]
    f1 = s.shape[1]
    f2 = w2a.shape[1]
    p3 = p.reshape(ng, LANE, f1a)
    grid = (n // tm,)
    return pl.pallas_call(
        _conv1_kernel,
        out_shape=(jax.ShapeDtypeStruct((n, f2), jnp.bfloat16),
                   jax.ShapeDtypeStruct((n, f2), jnp.float32),
                   jax.ShapeDtypeStruct((n, 1), jnp.float32)),
        grid=grid,
        in_specs=[
            pl.BlockSpec((ng, tm, LANE), lambda i: (0, i, 0)),
            pl.BlockSpec((ng, LANE, f1a), lambda i: (0, 0, 0)),
            pl.BlockSpec((tm, f1a), lambda i: (i, 0)),
            pl.BlockSpec((tm, f1), lambda i: (i, 0)),
            pl.BlockSpec((f1, f2), lambda i: (0, 0)),
            pl.BlockSpec((f1, f2), lambda i: (0, 0)),
            pl.BlockSpec((1, f2), lambda i: (0, 0)),
        ],
        out_specs=(pl.BlockSpec((tm, f2), lambda i: (i, 0)),
                   pl.BlockSpec((tm, f2), lambda i: (i, 0)),
                   pl.BlockSpec((tm, 1), lambda i: (i, 0))),
        compiler_params=_compiler_params(("parallel",)),
    )(a3, p3, p, s, w2a, w2s, b2)


def _conv2_mlp(a3, q, t, dinv, wl1, bl1, wl2, bl2, wl3, bl3, *, tm, kk):
    ng, n, _ = a3.shape
    f2 = q.shape[1]
    l1 = wl1.shape[1]
    l2 = wl2.shape[1]
    l3 = wl3.shape[1]
    q3 = q.reshape(ng, LANE, f2)
    grid = (n // tm,)
    return pl.pallas_call(
        _conv2_mlp_kernel,
        out_shape=jax.ShapeDtypeStruct((n, l3), jnp.float32),
        grid=grid,
        in_specs=[
            pl.BlockSpec((ng, tm, LANE), lambda i: (0, i, 0)),
            pl.BlockSpec((ng, LANE, f2), lambda i: (0, 0, 0)),
            pl.BlockSpec((tm, f2), lambda i: (i, 0)),
            pl.BlockSpec((tm, f2), lambda i: (i, 0)),
            pl.BlockSpec((tm, 1), lambda i: (i, 0)),
            pl.BlockSpec((f2, l1), lambda i: (0, 0)),
            pl.BlockSpec((1, l1), lambda i: (0, 0)),
            pl.BlockSpec((l1, l2), lambda i: (0, 0)),
            pl.BlockSpec((1, l2), lambda i: (0, 0)),
            pl.BlockSpec((l2, l3), lambda i: (0, 0)),
            pl.BlockSpec((1, l3), lambda i: (0, 0)),
        ],
        out_specs=pl.BlockSpec((tm, l3), lambda i: (i, 0)),
        compiler_params=_compiler_params(("parallel",)),
    )(a3, q3, q, t, dinv, wl1, bl1, wl2, bl2, wl3, bl3)


def kernel(x, edge_index, ws1, wa1, b1, ws2, wa2, b2,
           wl1, bl1, wl2, bl2, wl3, bl3):
    n = x.shape[0]
    ng = n // LANE
    tm, kk = 512, 64

    src, dst = edge_index[0], edge_index[1]
    not_self = src != dst

    lin = ((src >> 7) * (n * LANE)
           + (dst >> 3) * 1024
           + (dst & 7) * LANE
           + (src & 127))
    a3 = jnp.zeros((ng * n * LANE,), jnp.float32).at[lin].add(
        not_self.astype(jnp.float32)).reshape(ng, n, LANE)

    p, s = _proj(x, wa1, ws1, b1, tm=tm)
    q, t, dinv = _conv1(a3, p, s, wa2, ws2, b2, tm=tm, kk=kk)
    out = _conv2_mlp(a3, q, t, dinv, wl1, bl1, wl2, bl2, wl3, bl3,
                     tm=tm, kk=kk)
    return out[:, 0]


# speedup vs baseline: 2.4595x; 1.0096x over previous
import jax
import jax.numpy as jnp
from jax.experimental import pallas as pl
from jax.experimental.pallas import tpu as pltpu

LANE = 128


def _compiler_params(sem):
    return pltpu.CompilerParams(
        dimension_semantics=sem,
        vmem_limit_bytes=64 * 1024 * 1024,
    )


def _proj_kernel(x_ref, wa_ref, ws_ref, b_ref, p_ref, s_ref):
    xb = x_ref[...].astype(jnp.bfloat16)
    pb = jnp.dot(xb, wa_ref[...],
                 preferred_element_type=jnp.float32).astype(jnp.bfloat16)
    tm = pb.shape[0]
    ones_col = (jax.lax.broadcasted_iota(jnp.int32, (tm, LANE), 1)
                == 0).astype(jnp.bfloat16)
    p_ref[...] = jnp.concatenate([pb, ones_col], axis=1)
    s_ref[...] = jnp.dot(xb, ws_ref[...],
                         preferred_element_type=jnp.float32) + b_ref[...]


def _agg_dot(a_ref, pe3_ref, po3_ref):
    kk = a_ref.shape[0]
    packed = jnp.concatenate([a_ref[j] for j in range(kk)], axis=1)
    c_odd = jnp.floor(packed * (1.0 / 65536.0))
    c_even = packed - c_odd * 65536.0
    rhs_e = jnp.concatenate([pe3_ref[j] for j in range(kk)], axis=0)
    rhs_o = jnp.concatenate([po3_ref[j] for j in range(kk)], axis=0)
    return (jnp.dot(c_even.astype(jnp.bfloat16), rhs_e,
                    preferred_element_type=jnp.float32)
            + jnp.dot(c_odd.astype(jnp.bfloat16), rhs_o,
                      preferred_element_type=jnp.float32))


def _conv1_kernel(a_ref, pe3_ref, po3_ref, pself_ref, s_ref,
                  w2a_ref, w2s_ref, b2_ref, q_ref, t_ref, dinv_ref):
    f1 = s_ref.shape[1]
    acc = _agg_dot(a_ref, pe3_ref, po3_ref) + pself_ref[...].astype(jnp.float32)
    dinv = 1.0 / acc[:, f1:f1 + 1]
    h1 = jnp.maximum(s_ref[...] + acc[:, :f1] * dinv, 0.0)
    h1b = h1.astype(jnp.bfloat16)
    q_ref[...] = jnp.dot(h1b, w2a_ref[...],
                         preferred_element_type=jnp.float32).astype(jnp.bfloat16)
    t_ref[...] = jnp.dot(h1b, w2s_ref[...],
                         preferred_element_type=jnp.float32) + b2_ref[...]
    dinv_ref[...] = dinv


def _conv2_mlp_kernel(a_ref, qe3_ref, qo3_ref, qself_ref, t_ref, dinv_ref,
                      wl1_ref, bl1_ref, wl2_ref, bl2_ref, wl3_ref, bl3_ref,
                      o_ref):
    acc = _agg_dot(a_ref, qe3_ref, qo3_ref) + qself_ref[...].astype(jnp.float32)
    h2 = jnp.maximum(t_ref[...] + acc * dinv_ref[...], 0.0)
    s = jnp.dot(h2.astype(jnp.bfloat16), wl1_ref[...],
                preferred_element_type=jnp.float32) + bl1_ref[...]
    s = jnp.maximum(s, 0.0)
    s = jnp.dot(s.astype(jnp.bfloat16), wl2_ref[...],
                preferred_element_type=jnp.float32) + bl2_ref[...]
    s = jnp.maximum(s, 0.0)
    o_ref[...] = jnp.dot(s.astype(jnp.bfloat16), wl3_ref[...],
                         preferred_element_type=jnp.float32) + bl3_ref[...]


def _proj(x, wa1, ws1, b1, *, tm):
    n, f0 = x.shape
    f1 = wa1.shape[1]
    grid = (n // tm,)
    return pl.pallas_call(
        _proj_kernel,
        out_shape=(jax.ShapeDtypeStruct((n, f1 + LANE), jnp.bfloat16),
                   jax.ShapeDtypeStruct((n, f1), jnp.float32)),
        grid=grid,
        in_specs=[
            pl.BlockSpec((tm, f0), lambda i: (i, 0)),
            pl.BlockSpec((f0, f1), lambda i: (0, 0)),
            pl.BlockSpec((f0, f1), lambda i: (0, 0)),
            pl.BlockSpec((1, f1), lambda i: (0, 0)),
        ],
        out_specs=(pl.BlockSpec((tm, f1 + LANE), lambda i: (i, 0)),
                   pl.BlockSpec((tm, f1), lambda i: (i, 0))),
        compiler_params=_compiler_params(("parallel",)),
    )(x, wa1, ws1, b1)


def _conv1(a3, p, s, w2a, w2s, b2, *, tm, kk):
    ng, n, _ = a3.shape
    f1a = p.shape[1]
    f1 = s.shape[1]
    f2 = w2a.shape[1]
    pe3 = p[0::2].reshape(ng, LANE, f1a)
    po3 = p[1::2].reshape(ng, LANE, f1a)
    grid = (n // tm,)
    return pl.pallas_call(
        _conv1_kernel,
        out_shape=(jax.ShapeDtypeStruct((n, f2), jnp.bfloat16),
                   jax.ShapeDtypeStruct((n, f2), jnp.float32),
                   jax.ShapeDtypeStruct((n, 1), jnp.float32)),
        grid=grid,
        in_specs=[
            pl.BlockSpec((ng, tm, LANE), lambda i: (0, i, 0)),
            pl.BlockSpec((ng, LANE, f1a), lambda i: (0, 0, 0)),
            pl.BlockSpec((ng, LANE, f1a), lambda i: (0, 0, 0)),
            pl.BlockSpec((tm, f1a), lambda i: (i, 0)),
            pl.BlockSpec((tm, f1), lambda i: (i, 0)),
            pl.BlockSpec((f1, f2), lambda i: (0, 0)),
            pl.BlockSpec((f1, f2), lambda i: (0, 0)),
            pl.BlockSpec((1, f2), lambda i: (0, 0)),
        ],
        out_specs=(pl.BlockSpec((tm, f2), lambda i: (i, 0)),
                   pl.BlockSpec((tm, f2), lambda i: (i, 0)),
                   pl.BlockSpec((tm, 1), lambda i: (i, 0))),
        compiler_params=_compiler_params(("parallel",)),
    )(a3, pe3, po3, p, s, w2a, w2s, b2)


def _conv2_mlp(a3, q, t, dinv, wl1, bl1, wl2, bl2, wl3, bl3, *, tm, kk):
    ng, n, _ = a3.shape
    f2 = q.shape[1]
    l1 = wl1.shape[1]
    l2 = wl2.shape[1]
    l3 = wl3.shape[1]
    qe3 = q[0::2].reshape(ng, LANE, f2)
    qo3 = q[1::2].reshape(ng, LANE, f2)
    grid = (n // tm,)
    return pl.pallas_call(
        _conv2_mlp_kernel,
        out_shape=jax.ShapeDtypeStruct((n, l3), jnp.float32),
        grid=grid,
        in_specs=[
            pl.BlockSpec((ng, tm, LANE), lambda i: (0, i, 0)),
            pl.BlockSpec((ng, LANE, f2), lambda i: (0, 0, 0)),
            pl.BlockSpec((ng, LANE, f2), lambda i: (0, 0, 0)),
            pl.BlockSpec((tm, f2), lambda i: (i, 0)),
            pl.BlockSpec((tm, f2), lambda i: (i, 0)),
            pl.BlockSpec((tm, 1), lambda i: (i, 0)),
            pl.BlockSpec((f2, l1), lambda i: (0, 0)),
            pl.BlockSpec((1, l1), lambda i: (0, 0)),
            pl.BlockSpec((l1, l2), lambda i: (0, 0)),
            pl.BlockSpec((1, l2), lambda i: (0, 0)),
            pl.BlockSpec((l2, l3), lambda i: (0, 0)),
            pl.BlockSpec((1, l3), lambda i: (0, 0)),
        ],
        out_specs=pl.BlockSpec((tm, l3), lambda i: (i, 0)),
        compiler_params=_compiler_params(("parallel",)),
    )(a3, qe3, qo3, q, t, dinv, wl1, bl1, wl2, bl2, wl3, bl3)


def kernel(x, edge_index, ws1, wa1, b1, ws2, wa2, b2,
           wl1, bl1, wl2, bl2, wl3, bl3):
    n = x.shape[0]
    ng = n // (2 * LANE)
    tm, kk = 512, 64

    src, dst = edge_index[0], edge_index[1]
    not_self = src != dst

    lin = ((src >> 8) * (n * LANE)
           + (dst >> 3) * 1024
           + (dst & 7) * LANE
           + ((src >> 1) & 127))
    val = jnp.where((src & 1) == 1, 65536.0, 1.0) * not_self.astype(jnp.float32)
    a3 = jnp.zeros((ng * n * LANE,), jnp.float32).at[lin].add(
        val).reshape(ng, n, LANE)

    p, s = _proj(x, wa1, ws1, b1, tm=tm)
    q, t, dinv = _conv1(a3, p, s, wa2, ws2, b2, tm=tm, kk=kk)
    out = _conv2_mlp(a3, q, t, dinv, wl1, bl1, wl2, bl2, wl3, bl3,
                     tm=tm, kk=kk)
    return out[:, 0]


# speedup vs baseline: 3.0589x; 1.2437x over previous
import jax
import jax.numpy as jnp
from jax.experimental import pallas as pl
from jax.experimental.pallas import tpu as pltpu

LANE = 128


def _compiler_params(sem):
    return pltpu.CompilerParams(
        dimension_semantics=sem,
        vmem_limit_bytes=64 * 1024 * 1024,
    )


def _proj_kernel(x_ref, wa_ref, ws_ref, b_ref, p_ref, s_ref):
    xb = x_ref[...].astype(jnp.bfloat16)
    pb = jnp.dot(xb, wa_ref[...],
                 preferred_element_type=jnp.float32).astype(jnp.bfloat16)
    tm = pb.shape[0]
    ones_col = (jax.lax.broadcasted_iota(jnp.int32, (tm, LANE), 1)
                == 0).astype(jnp.bfloat16)
    p_ref[...] = jnp.concatenate([pb, ones_col], axis=1)
    s_ref[...] = jnp.dot(xb, ws_ref[...],
                         preferred_element_type=jnp.float32) + b_ref[...]


def _agg_dot(a_ref, pe3_ref, po3_ref):
    kk = a_ref.shape[0]
    packed = jnp.concatenate([a_ref[j] for j in range(kk)], axis=1)
    c_odd = jnp.floor(packed * (1.0 / 65536.0))
    c_even = packed - c_odd * 65536.0
    rhs_e = jnp.concatenate([pe3_ref[j] for j in range(kk)], axis=0)
    rhs_o = jnp.concatenate([po3_ref[j] for j in range(kk)], axis=0)
    return (jnp.dot(c_even.astype(jnp.bfloat16), rhs_e,
                    preferred_element_type=jnp.float32)
            + jnp.dot(c_odd.astype(jnp.bfloat16), rhs_o,
                      preferred_element_type=jnp.float32))


def _conv1_kernel(a_ref, pe3_ref, po3_ref, pself_ref, s_ref,
                  w2a_ref, w2s_ref, b2_ref, q_ref, t_ref, dinv_ref):
    f1 = s_ref.shape[1]
    acc = _agg_dot(a_ref, pe3_ref, po3_ref) + pself_ref[...].astype(jnp.float32)
    dinv = 1.0 / acc[:, f1:f1 + 1]
    h1 = jnp.maximum(s_ref[...] + acc[:, :f1] * dinv, 0.0)
    h1b = h1.astype(jnp.bfloat16)
    q_ref[...] = jnp.dot(h1b, w2a_ref[...],
                         preferred_element_type=jnp.float32).astype(jnp.bfloat16)
    t_ref[...] = jnp.dot(h1b, w2s_ref[...],
                         preferred_element_type=jnp.float32) + b2_ref[...]
    dinv_ref[...] = dinv


def _conv2_mlp_kernel(a_ref, qe3_ref, qo3_ref, qself_ref, t_ref, dinv_ref,
                      wl1_ref, bl1_ref, wl2_ref, bl2_ref, wl3_ref, bl3_ref,
                      o_ref):
    acc = _agg_dot(a_ref, qe3_ref, qo3_ref) + qself_ref[...].astype(jnp.float32)
    h2 = jnp.maximum(t_ref[...] + acc * dinv_ref[...], 0.0)
    s = jnp.dot(h2.astype(jnp.bfloat16), wl1_ref[...],
                preferred_element_type=jnp.float32) + bl1_ref[...]
    s = jnp.maximum(s, 0.0)
    s = jnp.dot(s.astype(jnp.bfloat16), wl2_ref[...],
                preferred_element_type=jnp.float32) + bl2_ref[...]
    s = jnp.maximum(s, 0.0)
    o_ref[...] = jnp.dot(s.astype(jnp.bfloat16), wl3_ref[...],
                         preferred_element_type=jnp.float32) + bl3_ref[...]


def _proj(x, wa1, ws1, b1, *, tm):
    n, f0 = x.shape
    f1 = wa1.shape[1]
    grid = (n // tm,)
    return pl.pallas_call(
        _proj_kernel,
        out_shape=(jax.ShapeDtypeStruct((n, f1 + LANE), jnp.bfloat16),
                   jax.ShapeDtypeStruct((n, f1), jnp.float32)),
        grid=grid,
        in_specs=[
            pl.BlockSpec((tm, f0), lambda i: (i, 0)),
            pl.BlockSpec((f0, f1), lambda i: (0, 0)),
            pl.BlockSpec((f0, f1), lambda i: (0, 0)),
            pl.BlockSpec((1, f1), lambda i: (0, 0)),
        ],
        out_specs=(pl.BlockSpec((tm, f1 + LANE), lambda i: (i, 0)),
                   pl.BlockSpec((tm, f1), lambda i: (i, 0))),
        compiler_params=_compiler_params(("parallel",)),
    )(x, wa1, ws1, b1)


def _conv1(a3, p, s, w2a, w2s, b2, *, tm, kk):
    ng, n, _ = a3.shape
    f1a = p.shape[1]
    f1 = s.shape[1]
    f2 = w2a.shape[1]
    p3 = p.reshape(2 * ng, LANE, f1a)
    grid = (n // tm,)
    return pl.pallas_call(
        _conv1_kernel,
        out_shape=(jax.ShapeDtypeStruct((n, f2), jnp.bfloat16),
                   jax.ShapeDtypeStruct((n, f2), jnp.float32),
                   jax.ShapeDtypeStruct((n, 1), jnp.float32)),
        grid=grid,
        in_specs=[
            pl.BlockSpec((ng, tm, LANE), lambda i: (0, i, 0)),
            pl.BlockSpec((ng, LANE, f1a), lambda i: (0, 0, 0)),
            pl.BlockSpec((ng, LANE, f1a), lambda i: (1, 0, 0)),
            pl.BlockSpec((tm, f1a), lambda i: (i, 0)),
            pl.BlockSpec((tm, f1), lambda i: (i, 0)),
            pl.BlockSpec((f1, f2), lambda i: (0, 0)),
            pl.BlockSpec((f1, f2), lambda i: (0, 0)),
            pl.BlockSpec((1, f2), lambda i: (0, 0)),
        ],
        out_specs=(pl.BlockSpec((tm, f2), lambda i: (i, 0)),
                   pl.BlockSpec((tm, f2), lambda i: (i, 0)),
                   pl.BlockSpec((tm, 1), lambda i: (i, 0))),
        compiler_params=_compiler_params(("parallel",)),
    )(a3, p3, p3, p, s, w2a, w2s, b2)


def _conv2_mlp(a3, q, t, dinv, wl1, bl1, wl2, bl2, wl3, bl3, *, tm, kk):
    ng, n, _ = a3.shape
    f2 = q.shape[1]
    l1 = wl1.shape[1]
    l2 = wl2.shape[1]
    l3 = wl3.shape[1]
    q3 = q.reshape(2 * ng, LANE, f2)
    grid = (n // tm,)
    return pl.pallas_call(
        _conv2_mlp_kernel,
        out_shape=jax.ShapeDtypeStruct((n, l3), jnp.float32),
        grid=grid,
        in_specs=[
            pl.BlockSpec((ng, tm, LANE), lambda i: (0, i, 0)),
            pl.BlockSpec((ng, LANE, f2), lambda i: (0, 0, 0)),
            pl.BlockSpec((ng, LANE, f2), lambda i: (1, 0, 0)),
            pl.BlockSpec((tm, f2), lambda i: (i, 0)),
            pl.BlockSpec((tm, f2), lambda i: (i, 0)),
            pl.BlockSpec((tm, 1), lambda i: (i, 0)),
            pl.BlockSpec((f2, l1), lambda i: (0, 0)),
            pl.BlockSpec((1, l1), lambda i: (0, 0)),
            pl.BlockSpec((l1, l2), lambda i: (0, 0)),
            pl.BlockSpec((1, l2), lambda i: (0, 0)),
            pl.BlockSpec((l2, l3), lambda i: (0, 0)),
            pl.BlockSpec((1, l3), lambda i: (0, 0)),
        ],
        out_specs=pl.BlockSpec((tm, l3), lambda i: (i, 0)),
        compiler_params=_compiler_params(("parallel",)),
    )(a3, q3, q3, q, t, dinv, wl1, bl1, wl2, bl2, wl3, bl3)


def kernel(x, edge_index, ws1, wa1, b1, ws2, wa2, b2,
           wl1, bl1, wl2, bl2, wl3, bl3):
    n = x.shape[0]
    ng = n // (2 * LANE)
    tm, kk = 512, 64

    src, dst = edge_index[0], edge_index[1]
    not_self = src != dst

    half = n // 2
    lin = (((src & (half - 1)) >> 7) * (n * LANE)
           + (dst >> 3) * 1024
           + (dst & 7) * LANE
           + (src & 127))
    val = jnp.where(src >= half, 65536.0, 1.0) * not_self.astype(jnp.float32)
    a3 = jnp.zeros((ng * n * LANE,), jnp.float32).at[lin].add(
        val).reshape(ng, n, LANE)

    p, s = _proj(x, wa1, ws1, b1, tm=tm)
    q, t, dinv = _conv1(a3, p, s, wa2, ws2, b2, tm=tm, kk=kk)
    out = _conv2_mlp(a3, q, t, dinv, wl1, bl1, wl2, bl2, wl3, bl3,
                     tm=tm, kk=kk)
    return out[:, 0]


# speedup vs baseline: 3.1553x; 1.0315x over previous
import jax
import jax.numpy as jnp
from jax.experimental import pallas as pl
from jax.experimental.pallas import tpu as pltpu

LANE = 128


def _compiler_params(sem):
    return pltpu.CompilerParams(
        dimension_semantics=sem,
        vmem_limit_bytes=64 * 1024 * 1024,
    )


def _proj_kernel(x_ref, wa_ref, ws_ref, b_ref, p_ref, s_ref):
    xb = x_ref[...].astype(jnp.bfloat16)
    pb = jnp.dot(xb, wa_ref[...],
                 preferred_element_type=jnp.float32).astype(jnp.bfloat16)
    tm = pb.shape[0]
    ones_col = (jax.lax.broadcasted_iota(jnp.int32, (tm, LANE), 1)
                == 0).astype(jnp.bfloat16)
    p_ref[...] = jnp.concatenate([pb, ones_col], axis=1)
    s_ref[...] = jnp.dot(xb, ws_ref[...],
                         preferred_element_type=jnp.float32) + b_ref[...]


def _agg_dot(a_ref, p0_ref, p1_ref, p2_ref, p3_ref):
    kk = a_ref.shape[0]
    packed = jnp.concatenate([a_ref[j] for j in range(kk)], axis=1)
    c3 = jnp.floor(packed * (1.0 / 16777216.0))
    r = packed - c3 * 16777216.0
    c2 = jnp.floor(r * (1.0 / 65536.0))
    r = r - c2 * 65536.0
    c1 = jnp.floor(r * (1.0 / 256.0))
    c0 = r - c1 * 256.0
    out = None
    for c, p_ref in ((c0, p0_ref), (c1, p1_ref), (c2, p2_ref), (c3, p3_ref)):
        rhs = jnp.concatenate([p_ref[j] for j in range(kk)], axis=0)
        d = jnp.dot(c.astype(jnp.bfloat16), rhs,
                    preferred_element_type=jnp.float32)
        out = d if out is None else out + d
    return out


def _conv1_kernel(a_ref, p0_ref, p1_ref, p2_ref, p3_ref, pself_ref, s_ref,
                  w2a_ref, w2s_ref, b2_ref, q_ref, t_ref, dinv_ref):
    f1 = s_ref.shape[1]
    acc = (_agg_dot(a_ref, p0_ref, p1_ref, p2_ref, p3_ref)
           + pself_ref[...].astype(jnp.float32))
    dinv = 1.0 / acc[:, f1:f1 + 1]
    h1 = jnp.maximum(s_ref[...] + acc[:, :f1] * dinv, 0.0)
    h1b = h1.astype(jnp.bfloat16)
    q_ref[...] = jnp.dot(h1b, w2a_ref[...],
                         preferred_element_type=jnp.float32).astype(jnp.bfloat16)
    t_ref[...] = jnp.dot(h1b, w2s_ref[...],
                         preferred_element_type=jnp.float32) + b2_ref[...]
    dinv_ref[...] = dinv


def _conv2_mlp_kernel(a_ref, q0_ref, q1_ref, q2_ref, q3_ref,
                      qself_ref, t_ref, dinv_ref,
                      wl1_ref, bl1_ref, wl2_ref, bl2_ref, wl3_ref, bl3_ref,
                      o_ref):
    acc = (_agg_dot(a_ref, q0_ref, q1_ref, q2_ref, q3_ref)
           + qself_ref[...].astype(jnp.float32))
    h2 = jnp.maximum(t_ref[...] + acc * dinv_ref[...], 0.0)
    s = jnp.dot(h2.astype(jnp.bfloat16), wl1_ref[...],
                preferred_element_type=jnp.float32) + bl1_ref[...]
    s = jnp.maximum(s, 0.0)
    s = jnp.dot(s.astype(jnp.bfloat16), wl2_ref[...],
                preferred_element_type=jnp.float32) + bl2_ref[...]
    s = jnp.maximum(s, 0.0)
    o_ref[...] = jnp.dot(s.astype(jnp.bfloat16), wl3_ref[...],
                         preferred_element_type=jnp.float32) + bl3_ref[...]


def _proj(x, wa1, ws1, b1, *, tm):
    n, f0 = x.shape
    f1 = wa1.shape[1]
    grid = (n // tm,)
    return pl.pallas_call(
        _proj_kernel,
        out_shape=(jax.ShapeDtypeStruct((n, f1 + LANE), jnp.bfloat16),
                   jax.ShapeDtypeStruct((n, f1), jnp.float32)),
        grid=grid,
        in_specs=[
            pl.BlockSpec((tm, f0), lambda i: (i, 0)),
            pl.BlockSpec((f0, f1), lambda i: (0, 0)),
            pl.BlockSpec((f0, f1), lambda i: (0, 0)),
            pl.BlockSpec((1, f1), lambda i: (0, 0)),
        ],
        out_specs=(pl.BlockSpec((tm, f1 + LANE), lambda i: (i, 0)),
                   pl.BlockSpec((tm, f1), lambda i: (i, 0))),
        compiler_params=_compiler_params(("parallel",)),
    )(x, wa1, ws1, b1)


def _conv1(a3, p, s, w2a, w2s, b2, *, tm, kk):
    ng, n, _ = a3.shape
    f1a = p.shape[1]
    f1 = s.shape[1]
    f2 = w2a.shape[1]
    p4 = p.reshape(4 * ng, LANE, f1a)
    grid = (n // tm,)
    return pl.pallas_call(
        _conv1_kernel,
        out_shape=(jax.ShapeDtypeStruct((n, f2), jnp.bfloat16),
                   jax.ShapeDtypeStruct((n, f2), jnp.float32),
                   jax.ShapeDtypeStruct((n, 1), jnp.float32)),
        grid=grid,
        in_specs=[
            pl.BlockSpec((ng, tm, LANE), lambda i: (0, i, 0)),
            pl.BlockSpec((ng, LANE, f1a), lambda i: (0, 0, 0)),
            pl.BlockSpec((ng, LANE, f1a), lambda i: (1, 0, 0)),
            pl.BlockSpec((ng, LANE, f1a), lambda i: (2, 0, 0)),
            pl.BlockSpec((ng, LANE, f1a), lambda i: (3, 0, 0)),
            pl.BlockSpec((tm, f1a), lambda i: (i, 0)),
            pl.BlockSpec((tm, f1), lambda i: (i, 0)),
            pl.BlockSpec((f1, f2), lambda i: (0, 0)),
            pl.BlockSpec((f1, f2), lambda i: (0, 0)),
            pl.BlockSpec((1, f2), lambda i: (0, 0)),
        ],
        out_specs=(pl.BlockSpec((tm, f2), lambda i: (i, 0)),
                   pl.BlockSpec((tm, f2), lambda i: (i, 0)),
                   pl.BlockSpec((tm, 1), lambda i: (i, 0))),
        compiler_params=_compiler_params(("parallel",)),
    )(a3, p4, p4, p4, p4, p, s, w2a, w2s, b2)


def _conv2_mlp(a3, q, t, dinv, wl1, bl1, wl2, bl2, wl3, bl3, *, tm, kk):
    ng, n, _ = a3.shape
    f2 = q.shape[1]
    l1 = wl1.shape[1]
    l2 = wl2.shape[1]
    l3 = wl3.shape[1]
    q4 = q.reshape(4 * ng, LANE, f2)
    grid = (n // tm,)
    return pl.pallas_call(
        _conv2_mlp_kernel,
        out_shape=jax.ShapeDtypeStruct((n, l3), jnp.float32),
        grid=grid,
        in_specs=[
            pl.BlockSpec((ng, tm, LANE), lambda i: (0, i, 0)),
            pl.BlockSpec((ng, LANE, f2), lambda i: (0, 0, 0)),
            pl.BlockSpec((ng, LANE, f2), lambda i: (1, 0, 0)),
            pl.BlockSpec((ng, LANE, f2), lambda i: (2, 0, 0)),
            pl.BlockSpec((ng, LANE, f2), lambda i: (3, 0, 0)),
            pl.BlockSpec((tm, f2), lambda i: (i, 0)),
            pl.BlockSpec((tm, f2), lambda i: (i, 0)),
            pl.BlockSpec((tm, 1), lambda i: (i, 0)),
            pl.BlockSpec((f2, l1), lambda i: (0, 0)),
            pl.BlockSpec((1, l1), lambda i: (0, 0)),
            pl.BlockSpec((l1, l2), lambda i: (0, 0)),
            pl.BlockSpec((1, l2), lambda i: (0, 0)),
            pl.BlockSpec((l2, l3), lambda i: (0, 0)),
            pl.BlockSpec((1, l3), lambda i: (0, 0)),
        ],
        out_specs=pl.BlockSpec((tm, l3), lambda i: (i, 0)),
        compiler_params=_compiler_params(("parallel",)),
    )(a3, q4, q4, q4, q4, q, t, dinv, wl1, bl1, wl2, bl2, wl3, bl3)


def kernel(x, edge_index, ws1, wa1, b1, ws2, wa2, b2,
           wl1, bl1, wl2, bl2, wl3, bl3):
    n = x.shape[0]
    ng = n // (4 * LANE)
    tm, kk = 512, 64

    src, dst = edge_index[0], edge_index[1]
    not_self = src != dst

    quarter = n // 4
    lin = (((src & (quarter - 1)) >> 7) * (n * LANE)
           + (dst >> 3) * 1024
           + (dst & 7) * LANE
           + (src & 127))
    val = (jnp.left_shift(1, 8 * (src // quarter)).astype(jnp.float32)
           * not_self.astype(jnp.float32))
    a3 = jnp.zeros((ng * n * LANE,), jnp.float32).at[lin].add(
        val).reshape(ng, n, LANE)

    p, s = _proj(x, wa1, ws1, b1, tm=tm)
    q, t, dinv = _conv1(a3, p, s, wa2, ws2, b2, tm=tm, kk=kk)
    out = _conv2_mlp(a3, q, t, dinv, wl1, bl1, wl2, bl2, wl3, bl3,
                     tm=tm, kk=kk)
    return out[:, 0]


# speedup vs baseline: 3.1644x; 1.0029x over previous
import jax
import jax.numpy as jnp
from jax.experimental import pallas as pl
from jax.experimental.pallas import tpu as pltpu

LANE = 128


def _compiler_params(sem):
    return pltpu.CompilerParams(
        dimension_semantics=sem,
        vmem_limit_bytes=64 * 1024 * 1024,
    )


def _proj_kernel(x_ref, wa_ref, ws_ref, b_ref, p_ref, s_ref):
    xb = x_ref[...].astype(jnp.bfloat16)
    pb = jnp.dot(xb, wa_ref[...],
                 preferred_element_type=jnp.float32).astype(jnp.bfloat16)
    tm = pb.shape[0]
    ones_col = (jax.lax.broadcasted_iota(jnp.int32, (tm, LANE), 1)
                == 0).astype(jnp.bfloat16)
    p_ref[...] = jnp.concatenate([pb, ones_col], axis=1)
    s_ref[...] = jnp.dot(xb, ws_ref[...],
                         preferred_element_type=jnp.float32) + b_ref[...]


def _agg_dot(a_ref, p0_ref, p1_ref, p2_ref, p3_ref):
    kk = a_ref.shape[0]
    packed = jnp.concatenate([a_ref[j] for j in range(kk)], axis=1)
    c3 = jnp.floor(packed * (1.0 / 262144.0))
    r = packed - c3 * 262144.0
    c2 = jnp.floor(r * (1.0 / 4096.0))
    r = r - c2 * 4096.0
    c1 = jnp.floor(r * (1.0 / 64.0))
    c0 = r - c1 * 64.0
    out = None
    for c, p_ref in ((c0, p0_ref), (c1, p1_ref), (c2, p2_ref), (c3, p3_ref)):
        rhs = jnp.concatenate([p_ref[j] for j in range(kk)], axis=0)
        d = jnp.dot(c.astype(jnp.bfloat16), rhs,
                    preferred_element_type=jnp.float32)
        out = d if out is None else out + d
    return out


def _conv1_kernel(a_ref, p0_ref, p1_ref, p2_ref, p3_ref, pself_ref, s_ref,
                  w2a_ref, w2s_ref, b2_ref, q_ref, t_ref, dinv_ref):
    f1 = s_ref.shape[1]
    acc = (_agg_dot(a_ref, p0_ref, p1_ref, p2_ref, p3_ref)
           + pself_ref[...].astype(jnp.float32))
    dinv = 1.0 / acc[:, f1:f1 + 1]
    h1 = jnp.maximum(s_ref[...] + acc[:, :f1] * dinv, 0.0)
    h1b = h1.astype(jnp.bfloat16)
    q_ref[...] = jnp.dot(h1b, w2a_ref[...],
                         preferred_element_type=jnp.float32).astype(jnp.bfloat16)
    t_ref[...] = jnp.dot(h1b, w2s_ref[...],
                         preferred_element_type=jnp.float32) + b2_ref[...]
    dinv_ref[...] = dinv


def _conv2_mlp_kernel(a_ref, q0_ref, q1_ref, q2_ref, q3_ref,
                      qself_ref, t_ref, dinv_ref,
                      wl1_ref, bl1_ref, wl2_ref, bl2_ref, wl3_ref, bl3_ref,
                      o_ref):
    acc = (_agg_dot(a_ref, q0_ref, q1_ref, q2_ref, q3_ref)
           + qself_ref[...].astype(jnp.float32))
    h2 = jnp.maximum(t_ref[...] + acc * dinv_ref[...], 0.0)
    s = jnp.dot(h2.astype(jnp.bfloat16), wl1_ref[...],
                preferred_element_type=jnp.float32) + bl1_ref[...]
    s = jnp.maximum(s, 0.0)
    s = jnp.dot(s.astype(jnp.bfloat16), wl2_ref[...],
                preferred_element_type=jnp.float32) + bl2_ref[...]
    s = jnp.maximum(s, 0.0)
    o_ref[...] = jnp.dot(s.astype(jnp.bfloat16), wl3_ref[...],
                         preferred_element_type=jnp.float32) + bl3_ref[...]


def _proj(x, wa1, ws1, b1, *, tm):
    n, f0 = x.shape
    f1 = wa1.shape[1]
    grid = (n // tm,)
    return pl.pallas_call(
        _proj_kernel,
        out_shape=(jax.ShapeDtypeStruct((n, f1 + LANE), jnp.bfloat16),
                   jax.ShapeDtypeStruct((n, f1), jnp.float32)),
        grid=grid,
        in_specs=[
            pl.BlockSpec((tm, f0), lambda i: (i, 0)),
            pl.BlockSpec((f0, f1), lambda i: (0, 0)),
            pl.BlockSpec((f0, f1), lambda i: (0, 0)),
            pl.BlockSpec((1, f1), lambda i: (0, 0)),
        ],
        out_specs=(pl.BlockSpec((tm, f1 + LANE), lambda i: (i, 0)),
                   pl.BlockSpec((tm, f1), lambda i: (i, 0))),
        compiler_params=_compiler_params(("parallel",)),
    )(x, wa1, ws1, b1)


def _conv1(a3, p, s, w2a, w2s, b2, *, tm, kk):
    ng, n, _ = a3.shape
    f1a = p.shape[1]
    f1 = s.shape[1]
    f2 = w2a.shape[1]
    p4 = p.reshape(4 * ng, LANE, f1a)
    grid = (n // tm,)
    return pl.pallas_call(
        _conv1_kernel,
        out_shape=(jax.ShapeDtypeStruct((n, f2), jnp.bfloat16),
                   jax.ShapeDtypeStruct((n, f2), jnp.float32),
                   jax.ShapeDtypeStruct((n, 1), jnp.float32)),
        grid=grid,
        in_specs=[
            pl.BlockSpec((ng, tm, LANE), lambda i: (0, i, 0)),
            pl.BlockSpec((ng, LANE, f1a), lambda i: (0, 0, 0)),
            pl.BlockSpec((ng, LANE, f1a), lambda i: (1, 0, 0)),
            pl.BlockSpec((ng, LANE, f1a), lambda i: (2, 0, 0)),
            pl.BlockSpec((ng, LANE, f1a), lambda i: (3, 0, 0)),
            pl.BlockSpec((tm, f1a), lambda i: (i, 0)),
            pl.BlockSpec((tm, f1), lambda i: (i, 0)),
            pl.BlockSpec((f1, f2), lambda i: (0, 0)),
            pl.BlockSpec((f1, f2), lambda i: (0, 0)),
            pl.BlockSpec((1, f2), lambda i: (0, 0)),
        ],
        out_specs=(pl.BlockSpec((tm, f2), lambda i: (i, 0)),
                   pl.BlockSpec((tm, f2), lambda i: (i, 0)),
                   pl.BlockSpec((tm, 1), lambda i: (i, 0))),
        compiler_params=_compiler_params(("parallel",)),
    )(a3, p4, p4, p4, p4, p, s, w2a, w2s, b2)


def _conv2_mlp(a3, q, t, dinv, wl1, bl1, wl2, bl2, wl3, bl3, *, tm, kk):
    ng, n, _ = a3.shape
    f2 = q.shape[1]
    l1 = wl1.shape[1]
    l2 = wl2.shape[1]
    l3 = wl3.shape[1]
    q4 = q.reshape(4 * ng, LANE, f2)
    grid = (n // tm,)
    return pl.pallas_call(
        _conv2_mlp_kernel,
        out_shape=jax.ShapeDtypeStruct((n, l3), jnp.float32),
        grid=grid,
        in_specs=[
            pl.BlockSpec((ng, tm, LANE), lambda i: (0, i, 0)),
            pl.BlockSpec((ng, LANE, f2), lambda i: (0, 0, 0)),
            pl.BlockSpec((ng, LANE, f2), lambda i: (1, 0, 0)),
            pl.BlockSpec((ng, LANE, f2), lambda i: (2, 0, 0)),
            pl.BlockSpec((ng, LANE, f2), lambda i: (3, 0, 0)),
            pl.BlockSpec((tm, f2), lambda i: (i, 0)),
            pl.BlockSpec((tm, f2), lambda i: (i, 0)),
            pl.BlockSpec((tm, 1), lambda i: (i, 0)),
            pl.BlockSpec((f2, l1), lambda i: (0, 0)),
            pl.BlockSpec((1, l1), lambda i: (0, 0)),
            pl.BlockSpec((l1, l2), lambda i: (0, 0)),
            pl.BlockSpec((1, l2), lambda i: (0, 0)),
            pl.BlockSpec((l2, l3), lambda i: (0, 0)),
            pl.BlockSpec((1, l3), lambda i: (0, 0)),
        ],
        out_specs=pl.BlockSpec((tm, l3), lambda i: (i, 0)),
        compiler_params=_compiler_params(("parallel",)),
    )(a3, q4, q4, q4, q4, q, t, dinv, wl1, bl1, wl2, bl2, wl3, bl3)


def kernel(x, edge_index, ws1, wa1, b1, ws2, wa2, b2,
           wl1, bl1, wl2, bl2, wl3, bl3):
    n = x.shape[0]
    ng = n // (4 * LANE)
    tm, kk = 512, 64

    src, dst = edge_index[0], edge_index[1]
    not_self = src != dst

    quarter = n // 4
    lin = (((src & (quarter - 1)) >> 7) * (n * LANE)
           + (dst >> 3) * 1024
           + (dst & 7) * LANE
           + (src & 127))
    val = (jnp.left_shift(1, 6 * (src // quarter)).astype(jnp.float32)
           * not_self.astype(jnp.float32))
    a3 = jnp.zeros((ng * n * LANE,), jnp.float32).at[lin].add(
        val).reshape(ng, n, LANE)

    p, s = _proj(x, wa1, ws1, b1, tm=tm)
    q, t, dinv = _conv1(a3, p, s, wa2, ws2, b2, tm=tm, kk=kk)
    out = _conv2_mlp(a3, q, t, dinv, wl1, bl1, wl2, bl2, wl3, bl3,
                     tm=tm, kk=kk)
    return out[:, 0]


# speedup vs baseline: 3.1726x; 1.0026x over previous
import jax
import jax.numpy as jnp
from jax.experimental import pallas as pl
from jax.experimental.pallas import tpu as pltpu

LANE = 128


def _compiler_params(sem):
    return pltpu.CompilerParams(
        dimension_semantics=sem,
        vmem_limit_bytes=64 * 1024 * 1024,
    )


def _proj_kernel(x_ref, wa_ref, ws_ref, b_ref, p_ref, s_ref):
    xb = x_ref[...].astype(jnp.bfloat16)
    pb = jnp.dot(xb, wa_ref[...],
                 preferred_element_type=jnp.float32).astype(jnp.bfloat16)
    tm = pb.shape[0]
    ones_col = (jax.lax.broadcasted_iota(jnp.int32, (tm, LANE), 1)
                == 0).astype(jnp.bfloat16)
    p_ref[...] = jnp.concatenate([pb, ones_col], axis=1)
    s_ref[...] = jnp.dot(xb, ws_ref[...],
                         preferred_element_type=jnp.float32) + b_ref[...]


def _agg_dot(a_ref, p0_ref, p1_ref, p2_ref, p3_ref):
    kk = a_ref.shape[0]
    packed = jnp.concatenate([a_ref[j] for j in range(kk)], axis=1)
    c3 = jnp.floor(packed * (1.0 / 262144.0))
    r = packed - c3 * 262144.0
    c2 = jnp.floor(r * (1.0 / 4096.0))
    r = r - c2 * 4096.0
    c1 = jnp.floor(r * (1.0 / 64.0))
    c0 = r - c1 * 64.0
    out = None
    for c, p_ref in ((c0, p0_ref), (c1, p1_ref), (c2, p2_ref), (c3, p3_ref)):
        rhs = jnp.concatenate([p_ref[j] for j in range(kk)], axis=0)
        d = jnp.dot(c.astype(jnp.bfloat16), rhs,
                    preferred_element_type=jnp.float32)
        out = d if out is None else out + d
    return out


def _conv1_kernel(a_ref, p0_ref, p1_ref, p2_ref, p3_ref, pself_ref, s_ref,
                  w2a_ref, w2s_ref, b2_ref, q_ref, t_ref, dinv_ref):
    f1 = s_ref.shape[1]
    acc = (_agg_dot(a_ref, p0_ref, p1_ref, p2_ref, p3_ref)
           + pself_ref[...].astype(jnp.float32))
    dinv = 1.0 / acc[:, f1:f1 + 1]
    h1 = jnp.maximum(s_ref[...] + acc[:, :f1] * dinv, 0.0)
    h1b = h1.astype(jnp.bfloat16)
    q_ref[...] = jnp.dot(h1b, w2a_ref[...],
                         preferred_element_type=jnp.float32).astype(jnp.bfloat16)
    t_ref[...] = jnp.dot(h1b, w2s_ref[...],
                         preferred_element_type=jnp.float32) + b2_ref[...]
    dinv_ref[...] = dinv


def _conv2_mlp_kernel(a_ref, q0_ref, q1_ref, q2_ref, q3_ref,
                      qself_ref, t_ref, dinv_ref,
                      wl1_ref, bl1_ref, wl2_ref, bl2_ref, wl3_ref, bl3_ref,
                      o_ref):
    acc = (_agg_dot(a_ref, q0_ref, q1_ref, q2_ref, q3_ref)
           + qself_ref[...].astype(jnp.float32))
    h2 = jnp.maximum(t_ref[...] + acc * dinv_ref[...], 0.0)
    s = jnp.dot(h2.astype(jnp.bfloat16), wl1_ref[...],
                preferred_element_type=jnp.float32) + bl1_ref[...]
    s = jnp.maximum(s, 0.0)
    s = jnp.dot(s.astype(jnp.bfloat16), wl2_ref[...],
                preferred_element_type=jnp.float32) + bl2_ref[...]
    s = jnp.maximum(s, 0.0)
    o_ref[...] = jnp.dot(s.astype(jnp.bfloat16), wl3_ref[...],
                         preferred_element_type=jnp.float32) + bl3_ref[...]


def _proj(x, wa1, ws1, b1, *, tm):
    n, f0 = x.shape
    f1 = wa1.shape[1]
    grid = (n // tm,)
    return pl.pallas_call(
        _proj_kernel,
        out_shape=(jax.ShapeDtypeStruct((n, f1 + LANE), jnp.bfloat16),
                   jax.ShapeDtypeStruct((n, f1), jnp.float32)),
        grid=grid,
        in_specs=[
            pl.BlockSpec((tm, f0), lambda i: (i, 0)),
            pl.BlockSpec((f0, f1), lambda i: (0, 0)),
            pl.BlockSpec((f0, f1), lambda i: (0, 0)),
            pl.BlockSpec((1, f1), lambda i: (0, 0)),
        ],
        out_specs=(pl.BlockSpec((tm, f1 + LANE), lambda i: (i, 0)),
                   pl.BlockSpec((tm, f1), lambda i: (i, 0))),
        compiler_params=_compiler_params(("parallel",)),
    )(x, wa1, ws1, b1)


def _conv1(a3, p, s, w2a, w2s, b2, *, tm, kk):
    ng, n, _ = a3.shape
    f1a = p.shape[1]
    f1 = s.shape[1]
    f2 = w2a.shape[1]
    p4 = p.reshape(4 * ng, LANE, f1a)
    grid = (n // tm,)
    return pl.pallas_call(
        _conv1_kernel,
        out_shape=(jax.ShapeDtypeStruct((n, f2), jnp.bfloat16),
                   jax.ShapeDtypeStruct((n, f2), jnp.float32),
                   jax.ShapeDtypeStruct((n, 1), jnp.float32)),
        grid=grid,
        in_specs=[
            pl.BlockSpec((ng, tm, LANE), lambda i: (0, i, 0)),
            pl.BlockSpec((ng, LANE, f1a), lambda i: (0, 0, 0)),
            pl.BlockSpec((ng, LANE, f1a), lambda i: (1, 0, 0)),
            pl.BlockSpec((ng, LANE, f1a), lambda i: (2, 0, 0)),
            pl.BlockSpec((ng, LANE, f1a), lambda i: (3, 0, 0)),
            pl.BlockSpec((tm, f1a), lambda i: (i, 0)),
            pl.BlockSpec((tm, f1), lambda i: (i, 0)),
            pl.BlockSpec((f1, f2), lambda i: (0, 0)),
            pl.BlockSpec((f1, f2), lambda i: (0, 0)),
            pl.BlockSpec((1, f2), lambda i: (0, 0)),
        ],
        out_specs=(pl.BlockSpec((tm, f2), lambda i: (i, 0)),
                   pl.BlockSpec((tm, f2), lambda i: (i, 0)),
                   pl.BlockSpec((tm, 1), lambda i: (i, 0))),
        compiler_params=_compiler_params(("parallel",)),
    )(a3, p4, p4, p4, p4, p, s, w2a, w2s, b2)


def _conv2_mlp(a3, q, t, dinv, wl1, bl1, wl2, bl2, wl3, bl3, *, tm, kk):
    ng, n, _ = a3.shape
    f2 = q.shape[1]
    l1 = wl1.shape[1]
    l2 = wl2.shape[1]
    l3 = wl3.shape[1]
    q4 = q.reshape(4 * ng, LANE, f2)
    grid = (n // tm,)
    return pl.pallas_call(
        _conv2_mlp_kernel,
        out_shape=jax.ShapeDtypeStruct((n, l3), jnp.float32),
        grid=grid,
        in_specs=[
            pl.BlockSpec((ng, tm, LANE), lambda i: (0, i, 0)),
            pl.BlockSpec((ng, LANE, f2), lambda i: (0, 0, 0)),
            pl.BlockSpec((ng, LANE, f2), lambda i: (1, 0, 0)),
            pl.BlockSpec((ng, LANE, f2), lambda i: (2, 0, 0)),
            pl.BlockSpec((ng, LANE, f2), lambda i: (3, 0, 0)),
            pl.BlockSpec((tm, f2), lambda i: (i, 0)),
            pl.BlockSpec((tm, f2), lambda i: (i, 0)),
            pl.BlockSpec((tm, 1), lambda i: (i, 0)),
            pl.BlockSpec((f2, l1), lambda i: (0, 0)),
            pl.BlockSpec((1, l1), lambda i: (0, 0)),
            pl.BlockSpec((l1, l2), lambda i: (0, 0)),
            pl.BlockSpec((1, l2), lambda i: (0, 0)),
            pl.BlockSpec((l2, l3), lambda i: (0, 0)),
            pl.BlockSpec((1, l3), lambda i: (0, 0)),
        ],
        out_specs=pl.BlockSpec((tm, l3), lambda i: (i, 0)),
        compiler_params=_compiler_params(("parallel",)),
    )(a3, q4, q4, q4, q4, q, t, dinv, wl1, bl1, wl2, bl2, wl3, bl3)


def kernel(x, edge_index, ws1, wa1, b1, ws2, wa2, b2,
           wl1, bl1, wl2, bl2, wl3, bl3):
    n = x.shape[0]
    ng = n // (4 * LANE)
    tm, kk = 1024, 64

    src, dst = edge_index[0], edge_index[1]
    not_self = src != dst

    quarter = n // 4
    lin = (((src & (quarter - 1)) >> 7) * (n * LANE)
           + (dst >> 3) * 1024
           + (dst & 7) * LANE
           + (src & 127))
    val = (jnp.left_shift(1, 6 * (src // quarter)).astype(jnp.float32)
           * not_self.astype(jnp.float32))
    a3 = jnp.zeros((ng * n * LANE,), jnp.float32).at[lin].add(
        val).reshape(ng, n, LANE)

    p, s = _proj(x, wa1, ws1, b1, tm=tm)
    q, t, dinv = _conv1(a3, p, s, wa2, ws2, b2, tm=tm, kk=kk)
    out = _conv2_mlp(a3, q, t, dinv, wl1, bl1, wl2, bl2, wl3, bl3,
                     tm=tm, kk=kk)
    return out[:, 0]
